# baseline (speedup 1.0000x reference)
import jax
import jax.numpy as jnp
from jax import lax
from jax.experimental import pallas as pl
from jax.experimental.pallas import tpu as pltpu
from jax.experimental.pallas import tpu_sc as plsc

D_EMB = 64
EMB_SCALE = 8.0
NC = 2
NS = 16
NW = NC * NS
CHUNK = 128
LANES = 16


def _emb_body(x_hbm, w_hbm, out_hbm, idx_v, rows_v, sem):
    wid = lax.axis_index("s") * NC + lax.axis_index("c")
    chunks_w = idx_v.shape[0]
    base_chunk = wid * chunks_w
    pltpu.sync_copy(x_hbm.at[pl.ds(base_chunk, chunks_w)], idx_v)

    def do_chunk(j, carry):
        pltpu.async_copy(w_hbm.at[idx_v.at[j]], rows_v, sem).wait()

        def scale_row(r, c2):
            for l in range(D_EMB // LANES):
                sl = (r, pl.ds(LANES * l, LANES))
                rows_v[sl] = rows_v[sl] * EMB_SCALE
            return c2

        lax.fori_loop(0, CHUNK, scale_row, 0)
        pltpu.sync_copy(rows_v,
                        out_hbm.at[pl.ds((base_chunk + j) * CHUNK, CHUNK)])
        return carry

    lax.fori_loop(0, chunks_w, do_chunk, 0)


def _make_kernel(nb, d):
    chunks_w = nb // (NW * CHUNK)
    mesh = plsc.VectorSubcoreMesh(core_axis_name="c", subcore_axis_name="s")
    return pl.kernel(
        _emb_body,
        mesh=mesh,
        compiler_params=pltpu.CompilerParams(use_tc_tiling_on_sc=False),
        out_type=jax.ShapeDtypeStruct((nb, d), jnp.float32),
        scratch_types=[
            pltpu.VMEM((chunks_w, CHUNK), jnp.int32),
            pltpu.VMEM((CHUNK, d), jnp.float32),
            pltpu.SemaphoreType.DMA,
        ],
    )


def kernel(x, weight):
    b, s = x.shape
    nb = b * s
    d = weight.shape[1]
    xf = x.reshape(nb // CHUNK, CHUNK).astype(jnp.int32)
    out = _make_kernel(nb, d)(xf, weight)
    return out.reshape(b, s, d)


# speedup vs baseline: 1.2057x; 1.2057x over previous
import jax
import jax.numpy as jnp
from jax import lax
from jax.experimental import pallas as pl
from jax.experimental.pallas import tpu as pltpu
from jax.experimental.pallas import tpu_sc as plsc

D_EMB = 64
EMB_SCALE = 8.0
NC = 2
NS = 16
NW = NC * NS
CHUNK = 128
LANES = 16
NBUF = 4


def _emb_body(x_hbm, w_hbm, out_hbm, idx_v, rows_v, gsem, osem):
    wid = lax.axis_index("s") * NC + lax.axis_index("c")
    chunks_w = idx_v.shape[0]
    base_chunk = wid * chunks_w
    pltpu.sync_copy(x_hbm.at[pl.ds(base_chunk, chunks_w)], idx_v)

    def start_gather(k, b):
        pltpu.async_copy(w_hbm.at[idx_v.at[k]], rows_v.at[b], gsem.at[b])

    def wait_gather(b):
        pltpu.make_async_copy(out_hbm.at[pl.ds(0, CHUNK)], rows_v.at[b],
                              gsem.at[b]).wait()

    def start_out(j, b):
        pltpu.async_copy(rows_v.at[b],
                         out_hbm.at[pl.ds((base_chunk + j) * CHUNK, CHUNK)],
                         osem.at[b])

    def wait_out(b):
        pltpu.make_async_copy(rows_v.at[b], out_hbm.at[pl.ds(0, CHUNK)],
                              osem.at[b]).wait()

    def scale(b):
        def srow(r, c):
            for l in range(D_EMB // LANES):
                sl = (b, r, pl.ds(LANES * l, LANES))
                rows_v[sl] = rows_v[sl] * EMB_SCALE
            return c
        lax.fori_loop(0, CHUNK, srow, 0, unroll=4)

    start_gather(0, 0)
    start_gather(1, 1)
    start_gather(2, 2)
    wait_gather(0); scale(0); start_out(0, 0)
    start_gather(3, 3)
    wait_gather(1); scale(1); start_out(1, 1)
    wait_out(0); start_gather(4, 0)
    wait_gather(2); scale(2); start_out(2, 2)
    wait_out(1); start_gather(5, 1)
    wait_gather(3); scale(3); start_out(3, 3)

    def outer(o, c):
        j0 = o * NBUF
        for b in range(NBUF):
            j = j0 + b
            b2 = (b + 2) % NBUF
            wait_out(b2)
            start_gather(j + 2, b2)
            wait_gather(b)
            scale(b)
            start_out(j, b)
        return c
    lax.fori_loop(1, chunks_w // NBUF - 1, outer, 0)

    jt = chunks_w - NBUF
    wait_out(2); start_gather(jt + 2, 2)
    wait_gather(0); scale(0); start_out(jt + 0, 0)
    wait_out(3); start_gather(jt + 3, 3)
    wait_gather(1); scale(1); start_out(jt + 1, 1)
    wait_out(0)
    wait_gather(2); scale(2); start_out(jt + 2, 2)
    wait_out(1)
    wait_gather(3); scale(3); start_out(jt + 3, 3)
    wait_out(2)
    wait_out(3)


def _make_kernel(nb, d):
    chunks_w = nb // (NW * CHUNK)
    mesh = plsc.VectorSubcoreMesh(core_axis_name="c", subcore_axis_name="s")
    return pl.kernel(
        _emb_body,
        mesh=mesh,
        compiler_params=pltpu.CompilerParams(use_tc_tiling_on_sc=False),
        out_type=jax.ShapeDtypeStruct((nb, d), jnp.float32),
        scratch_types=[
            pltpu.VMEM((chunks_w, CHUNK), jnp.int32),
            pltpu.VMEM((NBUF, CHUNK, d), jnp.float32),
            pltpu.SemaphoreType.DMA((NBUF,)),
            pltpu.SemaphoreType.DMA((NBUF,)),
        ],
    )


def kernel(x, weight):
    b, s = x.shape
    nb = b * s
    d = weight.shape[1]
    xf = x.reshape(nb // CHUNK, CHUNK).astype(jnp.int32)
    out = _make_kernel(nb, d)(xf, weight)
    return out.reshape(b, s, d)
